# baseline (speedup 1.0000x reference)
import functools

import jax
import jax.numpy as jnp
from jax import lax
from jax.experimental import pallas as pl
from jax.experimental.pallas import tpu as pltpu
from jax.experimental.pallas import tpu_sc as plsc

NC = 2
NS = 16
NW = NC * NS

NBUF = 8


def kernel(x, token_table, pos_table):
    B, S = x.shape
    V, D = token_table.shape
    LV = D // 16
    BW = B // NW

    mesh = plsc.VectorSubcoreMesh(core_axis_name="c", subcore_axis_name="s")

    @functools.partial(
        pl.kernel,
        mesh=mesh,
        out_type=jax.ShapeDtypeStruct((B, S, D), jnp.float32),
        scratch_types=[
            pltpu.VMEM((BW, S), jnp.int32),
            pltpu.VMEM((S, D), jnp.float32),
            pltpu.VMEM((NBUF, S, D), jnp.float32),
            pltpu.SemaphoreType.DMA((NBUF,)),
            pltpu.SemaphoreType.DMA((NBUF,)),
        ],
        compiler_params=pltpu.CompilerParams(use_tc_tiling_on_sc=False),
    )
    def run(x_hbm, tok_hbm, pos_hbm, out_hbm, idx_v, pos_v, buf_v, gsem, ssem):
        wid = lax.axis_index("s") * NC + lax.axis_index("c")
        row0 = wid * BW
        pltpu.sync_copy(x_hbm.at[pl.ds(row0, BW)], idx_v)
        pltpu.sync_copy(pos_hbm.at[pl.ds(0, S)], pos_v)

        HALVES = ((0, 104), (104, 96))

        def start_gather(r, slot):
            for off, ln in HALVES:
                pltpu.async_copy(
                    tok_hbm.at[idx_v.at[r, pl.ds(off, ln)]],
                    buf_v.at[slot, pl.ds(off, ln)],
                    gsem.at[slot])

        def wait_gather(slot):
            for off, ln in HALVES:
                pltpu.make_async_copy(
                    tok_hbm.at[idx_v.at[0, pl.ds(0, ln)]],
                    buf_v.at[slot, pl.ds(off, ln)],
                    gsem.at[slot]).wait()

        def start_scatter(r, slot):
            pltpu.async_copy(buf_v.at[slot], out_hbm.at[row0 + r],
                             ssem.at[slot])

        def wait_scatter(slot):
            pltpu.make_async_copy(buf_v.at[slot], out_hbm.at[0],
                                  ssem.at[slot]).wait()

        for g in range(5):
            start_gather(g, g)

        @pl.loop(0, BW, step=NBUF)
        def _ring(r0):
            for b in range(NBUF):
                slot = b
                r = r0 + b
                wait_gather(slot)

                @pl.loop(0, S, unroll=4)
                def _add(s):
                    for k in range(LV):
                        pv = pos_v[s, pl.ds(k * 16, 16)]
                        plsc.addupdate(buf_v.at[slot, s, pl.ds(k * 16, 16)],
                                       pv)

                start_scatter(r, slot)

                nxt = r + 5
                nslot = (b + 5) % NBUF

                @pl.when(nxt < BW)
                def _():
                    @pl.when(nxt >= NBUF)
                    def _():
                        wait_scatter(nslot)

                    start_gather(nxt, nslot)

        for b in range(NBUF):
            wait_scatter((BW - NBUF + b) % NBUF)

    return run(x.astype(jnp.int32), token_table, pos_table)


# speedup vs baseline: 1.0024x; 1.0024x over previous
import functools

import jax
import jax.numpy as jnp
from jax import lax
from jax.experimental import pallas as pl
from jax.experimental.pallas import tpu as pltpu
from jax.experimental.pallas import tpu_sc as plsc

NC = 2
NS = 16
NW = NC * NS

NBUF = 4


def kernel(x, token_table, pos_table):
    B, S = x.shape
    V, D = token_table.shape
    LV = D // 16
    BW = B // NW

    mesh = plsc.VectorSubcoreMesh(core_axis_name="c", subcore_axis_name="s")

    @functools.partial(
        pl.kernel,
        mesh=mesh,
        out_type=jax.ShapeDtypeStruct((B, S, D), jnp.float32),
        scratch_types=[
            pltpu.VMEM((BW, S), jnp.int32),
            pltpu.VMEM((S, D), jnp.float32),
            pltpu.VMEM((NBUF, S, D), jnp.float32),
            pltpu.SemaphoreType.DMA((NBUF,)),
            pltpu.SemaphoreType.DMA((NBUF,)),
        ],
        compiler_params=pltpu.CompilerParams(use_tc_tiling_on_sc=False),
    )
    def run(x_hbm, tok_hbm, pos_hbm, out_hbm, idx_v, pos_v, buf_v, gsem, ssem):
        wid = lax.axis_index("s") * NC + lax.axis_index("c")
        row0 = wid * BW
        pltpu.sync_copy(x_hbm.at[pl.ds(row0, BW)], idx_v)
        pltpu.sync_copy(pos_hbm.at[pl.ds(0, S)], pos_v)

        HALVES = ((0, 104), (104, 96))

        def start_gather(r, slot):
            for off, ln in HALVES:
                pltpu.async_copy(
                    tok_hbm.at[idx_v.at[r, pl.ds(off, ln)]],
                    buf_v.at[slot, pl.ds(off, ln)],
                    gsem.at[slot])

        def wait_gather(slot):
            for off, ln in HALVES:
                pltpu.make_async_copy(
                    tok_hbm.at[idx_v.at[0, pl.ds(0, ln)]],
                    buf_v.at[slot, pl.ds(off, ln)],
                    gsem.at[slot]).wait()

        def start_scatter(r, slot):
            pltpu.async_copy(buf_v.at[slot], out_hbm.at[row0 + r],
                             ssem.at[slot])

        def wait_scatter(slot):
            pltpu.make_async_copy(buf_v.at[slot], out_hbm.at[0],
                                  ssem.at[slot]).wait()

        for g in range(3):
            start_gather(g, g)

        @pl.loop(0, BW, step=NBUF)
        def _ring(r0):
            for b in range(NBUF):
                slot = b
                r = r0 + b
                wait_gather(slot)

                @pl.loop(0, S, unroll=4)
                def _add(s):
                    for k in range(LV):
                        pv = pos_v[s, pl.ds(k * 16, 16)]
                        plsc.addupdate(buf_v.at[slot, s, pl.ds(k * 16, 16)],
                                       pv)

                start_scatter(r, slot)

                nxt = r + 3
                nslot = (b + 3) % NBUF

                @pl.when(nxt < BW)
                def _():
                    @pl.when(nxt >= NBUF)
                    def _():
                        wait_scatter(nslot)

                    start_gather(nxt, nslot)

        for b in range(NBUF):
            wait_scatter((BW - NBUF + b) % NBUF)

    return run(x.astype(jnp.int32), token_table, pos_table)
